# baseline (speedup 1.0000x reference)
import functools

import jax
import jax.numpy as jnp
from jax import lax
from jax.experimental import pallas as pl
from jax.experimental.pallas import tpu as pltpu
from jax.experimental.pallas import tpu_sc as plsc

NC = 2
NS = 16
L = 16
NW = NC * NS

B = 4096
S = 200
D = 64
DV = D // L

BPW = B // NW

CH0 = 128
CH1 = S - CH0


def _pool_body(x_hbm, emb_hbm, out_hbm, idx_v, rows_v, out_v, sem):
    wid = lax.axis_index("s") * NC + lax.axis_index("c")
    base = wid * BPW

    def body(b, carry):
        pltpu.sync_copy(x_hbm.at[base + b], idx_v)
        cp0 = pltpu.async_copy(
            emb_hbm.at[idx_v.at[pl.ds(0, CH0)]], rows_v.at[pl.ds(0, CH0)], sem)
        cp1 = pltpu.async_copy(
            emb_hbm.at[idx_v.at[pl.ds(CH0, CH1)]], rows_v.at[pl.ds(CH0, CH1)], sem)
        cp0.wait()
        cp1.wait()

        def rbody(r, accs):
            return tuple(
                accs[j] + rows_v[r, pl.ds(j * L, L)] for j in range(DV))

        accs = lax.fori_loop(
            0, S, rbody, tuple(jnp.zeros((L,), jnp.float32) for _ in range(DV)))
        scale = jnp.float32(1.0 / S)
        for j in range(DV):
            out_v[b, pl.ds(j * L, L)] = accs[j] * scale
        return carry

    lax.fori_loop(0, BPW, body, 0)
    pltpu.sync_copy(out_v, out_hbm.at[pl.ds(base, BPW)])


@functools.lru_cache(maxsize=1)
def _pool():
    return pl.kernel(
        _pool_body,
        out_type=jax.ShapeDtypeStruct((B, D), jnp.float32),
        mesh=plsc.VectorSubcoreMesh(
            core_axis_name="c", subcore_axis_name="s",
            num_cores=NC, num_subcores=NS),
        scratch_types=[
            pltpu.VMEM((S,), jnp.int32),
            pltpu.VMEM((S, D), jnp.float32),
            pltpu.VMEM((BPW, D), jnp.float32),
            pltpu.SemaphoreType.DMA,
        ],
        compiler_params=pltpu.CompilerParams(use_tc_tiling_on_sc=False),
    )


def _mlp_body(p_ref, w1t_ref, b1_ref, w2t_ref, b2_ref, w3t_ref, b3_ref, o_ref):
    h = jnp.maximum(
        jnp.dot(p_ref[...], w1t_ref[...], preferred_element_type=jnp.float32)
        + b1_ref[...], 0.0)
    h = jnp.maximum(
        jnp.dot(h, w2t_ref[...], preferred_element_type=jnp.float32)
        + b2_ref[...], 0.0)
    logits = (jnp.dot(h, w3t_ref[...], preferred_element_type=jnp.float32)
              + b3_ref[...])
    m = jnp.max(logits, axis=1, keepdims=True)
    e = jnp.exp(logits - m)
    o_ref[...] = e / jnp.sum(e, axis=1, keepdims=True)


def _mlp(pooled, w1t, b1p, w2t, b2p, w3t, b3p):
    return pl.pallas_call(
        _mlp_body,
        out_shape=jax.ShapeDtypeStruct((B, 128), jnp.float32),
    )(pooled, w1t, b1p, w2t, b2p, w3t, b3p)


def kernel(x, emb, w1, b1, w2, b2, w3, b3):
    x = x.astype(jnp.int32)
    pooled = _pool()(x, emb)

    w1t = w1.T
    b1p = b1.reshape(1, 128)
    w2t = jnp.zeros((128, 128), jnp.float32).at[:, :32].set(w2.T)
    b2p = jnp.zeros((1, 128), jnp.float32).at[0, :32].set(b2)
    w3t = jnp.zeros((128, 128), jnp.float32).at[:32, :10].set(w3.T)
    b3p = jnp.full((1, 128), -1e30, jnp.float32).at[0, :10].set(b3)

    out = _mlp(pooled, w1t, b1p, w2t, b2p, w3t, b3p)
    return out[:, :10]


# speedup vs baseline: 1.2443x; 1.2443x over previous
import functools

import jax
import jax.numpy as jnp
from jax import lax
from jax.experimental import pallas as pl
from jax.experimental.pallas import tpu as pltpu
from jax.experimental.pallas import tpu_sc as plsc

NC = 2
NS = 16
L = 16
NW = NC * NS

B = 4096
S = 200
D = 64
DV = D // L

BPW = B // NW

CH0 = 128
CH1 = S - CH0


R_UNROLL = 8


def _pool_body(x_hbm, emb_hbm, out_hbm, xv, buf0, buf1, out_v, sem0, sem1):
    wid = lax.axis_index("s") * NC + lax.axis_index("c")
    base = wid * BPW

    pltpu.sync_copy(x_hbm.at[pl.ds(base * S, BPW * S)], xv)

    bufs = (buf0, buf1)
    sems = (sem0, sem1)

    def copies(b, k):
        o = b * S
        return (
            pltpu.make_async_copy(
                emb_hbm.at[xv.at[pl.ds(o, CH0)]],
                bufs[k].at[pl.ds(0, CH0)], sems[k]),
            pltpu.make_async_copy(
                emb_hbm.at[xv.at[pl.ds(o + CH0, CH1)]],
                bufs[k].at[pl.ds(CH0, CH1)], sems[k]),
        )

    def issue(b, k):
        for c in copies(b, k):
            c.start()

    issue(0, 0)
    scale = jnp.float32(1.0 / S)

    @pl.loop(0, BPW, step=2)
    def _(b):
        for k in range(2):
            bb = b + k
            nxt = bb + 1

            @pl.when(nxt < BPW)
            def _():
                issue(nxt, (k + 1) % 2)

            for c in copies(bb, k):
                c.wait()

            buf = bufs[k]

            def rbody(r, accs):
                a = list(accs)
                for rr in range(R_UNROLL):
                    row = r * R_UNROLL + rr
                    for j in range(DV):
                        a[j] = a[j] + buf[row, pl.ds(j * L, L)]
                return tuple(a)

            accs = lax.fori_loop(
                0, S // R_UNROLL, rbody,
                tuple(jnp.zeros((L,), jnp.float32) for _ in range(DV)))
            for j in range(DV):
                out_v[bb, pl.ds(j * L, L)] = accs[j] * scale

    pltpu.sync_copy(out_v, out_hbm.at[pl.ds(base, BPW)])


@functools.lru_cache(maxsize=1)
def _pool():
    return pl.kernel(
        _pool_body,
        out_type=jax.ShapeDtypeStruct((B, D), jnp.float32),
        mesh=plsc.VectorSubcoreMesh(
            core_axis_name="c", subcore_axis_name="s",
            num_cores=NC, num_subcores=NS),
        scratch_types=[
            pltpu.VMEM((BPW * S,), jnp.int32),
            pltpu.VMEM((S, D), jnp.float32),
            pltpu.VMEM((S, D), jnp.float32),
            pltpu.VMEM((BPW, D), jnp.float32),
            pltpu.SemaphoreType.DMA,
            pltpu.SemaphoreType.DMA,
        ],
        compiler_params=pltpu.CompilerParams(use_tc_tiling_on_sc=False),
    )


def _mlp_body(p_ref, w1t_ref, b1_ref, w2t_ref, b2_ref, w3t_ref, b3_ref, o_ref):
    h = jnp.maximum(
        jnp.dot(p_ref[...], w1t_ref[...], preferred_element_type=jnp.float32)
        + b1_ref[...], 0.0)
    h = jnp.maximum(
        jnp.dot(h, w2t_ref[...], preferred_element_type=jnp.float32)
        + b2_ref[...], 0.0)
    logits = (jnp.dot(h, w3t_ref[...], preferred_element_type=jnp.float32)
              + b3_ref[...])
    m = jnp.max(logits, axis=1, keepdims=True)
    e = jnp.exp(logits - m)
    o_ref[...] = e / jnp.sum(e, axis=1, keepdims=True)


def _mlp(pooled, w1t, b1p, w2t, b2p, w3t, b3p):
    return pl.pallas_call(
        _mlp_body,
        out_shape=jax.ShapeDtypeStruct((B, 128), jnp.float32),
    )(pooled, w1t, b1p, w2t, b2p, w3t, b3p)


def kernel(x, emb, w1, b1, w2, b2, w3, b3):
    x = x.astype(jnp.int32).reshape(B * S)
    pooled = _pool()(x, emb)

    w1t = w1.T
    b1p = b1.reshape(1, 128)
    w2t = jnp.zeros((128, 128), jnp.float32).at[:, :32].set(w2.T)
    b2p = jnp.zeros((1, 128), jnp.float32).at[0, :32].set(b2)
    w3t = jnp.zeros((128, 128), jnp.float32).at[:32, :10].set(w3.T)
    b3p = jnp.full((1, 128), -1e30, jnp.float32).at[0, :10].set(b3)

    out = _mlp(pooled, w1t, b1p, w2t, b2p, w3t, b3p)
    return out[:, :10]
